# baseline (speedup 1.0000x reference)
import functools

import jax
import jax.numpy as jnp
from jax import lax
from jax.experimental import pallas as pl
from jax.experimental.pallas import tpu as pltpu
from jax.experimental.pallas import tpu_sc as plsc


def _make_x_gather(N, D, n_workers):
    rows_per_w = N // n_workers
    mesh = plsc.VectorSubcoreMesh(core_axis_name="c", subcore_axis_name="s")

    @functools.partial(
        pl.kernel,
        mesh=mesh,
        out_type=jax.ShapeDtypeStruct((N, D), jnp.float32),
        scratch_types=[
            pltpu.VMEM((rows_per_w,), jnp.int32),
            pltpu.VMEM((rows_per_w, D), jnp.float32),
            pltpu.SemaphoreType.DMA,
        ],
    )
    def gather(emb_hbm, idx_hbm, out_hbm, idx_v, rows_v, sem):
        n_cores = lax.axis_size("c")
        wid = lax.axis_index("s") * n_cores + lax.axis_index("c")
        base = wid * rows_per_w
        pltpu.sync_copy(idx_hbm.at[pl.ds(base, rows_per_w)], idx_v)
        pltpu.async_copy(emb_hbm.at[idx_v], rows_v, sem).wait()
        pltpu.sync_copy(rows_v, out_hbm.at[pl.ds(base, rows_per_w)])

    return gather


def _head_body(x_ref, w_ref, b_ref, out_ref):
    lb, V, B = out_ref.shape
    step = pl.program_id(0)
    for j in range(lb):
        acc = lax.dot_general(
            w_ref[...], x_ref[pl.ds((step * lb + j) * B, B), :],
            dimension_numbers=(((1,), (1,)), ((), ())),
            preferred_element_type=jnp.float32,
        )
        out_ref[j] = acc + b_ref[...]


def _make_head(B, L, D, V, lblock):
    return pl.pallas_call(
        _head_body,
        grid=(L // lblock,),
        in_specs=[
            pl.BlockSpec((L * B, D), lambda l: (0, 0)),
            pl.BlockSpec((V, D), lambda l: (0, 0)),
            pl.BlockSpec((V, 1), lambda l: (0, 0)),
        ],
        out_specs=pl.BlockSpec((lblock, V, B), lambda l: (l, 0, 0)),
        out_shape=jax.ShapeDtypeStruct((L, V, B), jnp.float32),
    )


def kernel(input_ids, emb_table, head_w, head_b):
    B, L = input_ids.shape
    V, D = emb_table.shape
    ids = input_ids.T.reshape(-1).astype(jnp.int32)
    N = B * L
    x = _make_x_gather(N, D, n_workers=32)(emb_table, ids)
    out_phys = _make_head(B, L, D, V, lblock=2)(x, head_w, head_b.reshape(V, 1))
    return out_phys.transpose(2, 0, 1)


# speedup vs baseline: 1.0099x; 1.0099x over previous
import functools

import jax
import jax.numpy as jnp
from jax import lax
from jax.experimental import pallas as pl
from jax.experimental.pallas import tpu as pltpu
from jax.experimental.pallas import tpu_sc as plsc


def _make_x_gather(N, D, n_workers):
    rows_per_w = N // n_workers
    mesh = plsc.VectorSubcoreMesh(core_axis_name="c", subcore_axis_name="s")

    @functools.partial(
        pl.kernel,
        mesh=mesh,
        out_type=jax.ShapeDtypeStruct((N, D), jnp.float32),
        scratch_types=[
            pltpu.VMEM((rows_per_w,), jnp.int32),
            pltpu.VMEM((rows_per_w, D), jnp.float32),
            pltpu.SemaphoreType.DMA,
        ],
    )
    def gather(emb_hbm, idx_hbm, out_hbm, idx_v, rows_v, sem):
        n_cores = lax.axis_size("c")
        wid = lax.axis_index("s") * n_cores + lax.axis_index("c")
        base = wid * rows_per_w
        pltpu.sync_copy(idx_hbm.at[pl.ds(base, rows_per_w)], idx_v)
        pltpu.async_copy(emb_hbm.at[idx_v], rows_v, sem).wait()
        pltpu.sync_copy(rows_v, out_hbm.at[pl.ds(base, rows_per_w)])

    return gather


def _head_body(x_ref, w_ref, b_ref, out_ref):
    lb, V, B = out_ref.shape
    for j in range(lb):
        acc = lax.dot_general(
            w_ref[...], x_ref[pl.ds(j * B, B), :],
            dimension_numbers=(((1,), (1,)), ((), ())),
            preferred_element_type=jnp.float32,
        )
        out_ref[j] = acc + b_ref[...]


def _make_head(B, L, D, V, lblock):
    return pl.pallas_call(
        _head_body,
        grid=(L // lblock,),
        in_specs=[
            pl.BlockSpec((lblock * B, D), lambda l: (l, 0)),
            pl.BlockSpec((V, D), lambda l: (0, 0)),
            pl.BlockSpec((V, 1), lambda l: (0, 0)),
        ],
        out_specs=pl.BlockSpec((lblock, V, B), lambda l: (l, 0, 0)),
        out_shape=jax.ShapeDtypeStruct((L, V, B), jnp.float32),
    )


def kernel(input_ids, emb_table, head_w, head_b):
    B, L = input_ids.shape
    V, D = emb_table.shape
    ids = input_ids.T.reshape(-1).astype(jnp.int32)
    N = B * L
    x = _make_x_gather(N, D, n_workers=32)(emb_table, ids)
    out_phys = _make_head(B, L, D, V, lblock=2)(x, head_w, head_b.reshape(V, 1))
    return out_phys.transpose(2, 0, 1)
